# baseline (speedup 1.0000x reference)
import functools

import jax
import jax.numpy as jnp
from jax import lax
from jax.experimental import pallas as pl
from jax.experimental.pallas import tpu as pltpu
from jax.experimental.pallas import tpu_sc as plsc

D_MODEL = 128
N_ROWS = 16384 * 200
NC, NS, L = 2, 16, 16
NW = NC * NS
ROWS_PER_W = N_ROWS // NW
CHUNK = 256
K = CHUNK // 128
N_CHUNKS = ROWS_PER_W // CHUNK


def _build_xidx(xidx, base):
    lane8 = lax.iota(jnp.int32, L) * 8
    for m in range(CHUNK // L):
        xidx[pl.ds(m * L, L)] = lane8 + ((base + m * L) * 8 + 7)


def _start_xgather(x_hbm, xidx, fstage, sem):
    for j in range(K):
        pltpu.async_copy(
            x_hbm.at[xidx.at[pl.ds(j * 128, 128)]],
            fstage.at[pl.ds(j * 128, 128)], sem)


def _wait_xgather(x_hbm, xidx, fstage, sem):
    for j in range(K):
        pltpu.make_async_copy(
            x_hbm.at[xidx.at[pl.ds(j * 128, 128)]],
            fstage.at[pl.ds(j * 128, 128)], sem).wait()


def _start_tgather(table_hbm, idxb, rowbuf, sem):
    for j in range(K):
        pltpu.async_copy(
            table_hbm.at[idxb.at[pl.ds(j * 128, 128)]],
            rowbuf.at[pl.ds(j * 128, 128)], sem)


def _wait_tgather(table_hbm, idxb, rowbuf, sem):
    for j in range(K):
        pltpu.make_async_copy(
            table_hbm.at[idxb.at[pl.ds(j * 128, 128)]],
            rowbuf.at[pl.ds(j * 128, 128)], sem).wait()


def _sc_lookup(x_hbm, table_hbm, out_hbm,
               xidx0, xidx1, fst0, fst1, idx0, idx1, row0, row1,
               semx0, semx1, semt0, semt1, semw0, semw1):
    wid = lax.axis_index("s") * NC + lax.axis_index("c")
    w_base = wid * ROWS_PER_W
    xidx = [xidx0, xidx1]
    fst = [fst0, fst1]
    idxb = [idx0, idx1]
    row = [row0, row1]
    semx = [semx0, semx1]
    semt = [semt0, semt1]
    semw = [semw0, semw1]

    def halfstep(g, p, first, last):
        base = w_base + g * CHUNK
        q = 1 - p
        _wait_xgather(x_hbm, xidx[p], fst[p], semx[p])
        for i in range(CHUNK // L):
            v = fst[p][pl.ds(i * L, L)]
            idxb[p][pl.ds(i * L, L)] = v.astype(jnp.int32)
        if not first:
            pltpu.make_async_copy(
                row[p], out_hbm.at[pl.ds(base - 2 * CHUNK, CHUNK)],
                semw[p]).wait()
        _start_tgather(table_hbm, idxb[p], row[p], semt[p])
        if not last:
            nbase = w_base + (g + 1) * CHUNK
            _build_xidx(xidx[q], nbase)
            _start_xgather(x_hbm, xidx[q], fst[q], semx[q])
        _wait_tgather(table_hbm, idxb[p], row[p], semt[p])
        pltpu.async_copy(row[p], out_hbm.at[pl.ds(base, CHUNK)], semw[p])

    _build_xidx(xidx[0], w_base)
    _start_xgather(x_hbm, xidx[0], fst[0], semx[0])

    def pair_body(t, carry):
        g0 = t * 2

        @pl.when(t == 0)
        def _():
            halfstep(g0, 0, True, False)
            halfstep(g0 + 1, 1, True, False)

        @pl.when(jnp.logical_and(t > 0, t < N_CHUNKS // 2 - 1))
        def _():
            halfstep(g0, 0, False, False)
            halfstep(g0 + 1, 1, False, False)

        @pl.when(t == N_CHUNKS // 2 - 1)
        def _():
            halfstep(g0, 0, False, False)
            halfstep(g0 + 1, 1, False, True)

        return carry

    lax.fori_loop(0, N_CHUNKS // 2, pair_body, 0)

    endb = w_base + N_CHUNKS * CHUNK
    pltpu.make_async_copy(
        row[0], out_hbm.at[pl.ds(endb - 2 * CHUNK, CHUNK)], semw[0]).wait()
    pltpu.make_async_copy(
        row[1], out_hbm.at[pl.ds(endb - CHUNK, CHUNK)], semw[1]).wait()


def kernel(x, table):
    x_flat = x.reshape(N_ROWS * 8)
    mesh = plsc.VectorSubcoreMesh(core_axis_name="c", subcore_axis_name="s")
    f = functools.partial(
        pl.kernel,
        mesh=mesh,
        out_type=jax.ShapeDtypeStruct((N_ROWS, D_MODEL), jnp.float32),
        scratch_types=[
            pltpu.VMEM((CHUNK,), jnp.int32),
            pltpu.VMEM((CHUNK,), jnp.int32),
            pltpu.VMEM((CHUNK,), jnp.float32),
            pltpu.VMEM((CHUNK,), jnp.float32),
            pltpu.VMEM((CHUNK,), jnp.int32),
            pltpu.VMEM((CHUNK,), jnp.int32),
            pltpu.VMEM((CHUNK, D_MODEL), jnp.float32),
            pltpu.VMEM((CHUNK, D_MODEL), jnp.float32),
            pltpu.SemaphoreType.DMA,
            pltpu.SemaphoreType.DMA,
            pltpu.SemaphoreType.DMA,
            pltpu.SemaphoreType.DMA,
            pltpu.SemaphoreType.DMA,
            pltpu.SemaphoreType.DMA,
        ],
    )(_sc_lookup)
    out = f(x_flat, table)
    return out.reshape(x.shape[0], x.shape[1], D_MODEL)


# speedup vs baseline: 8.6150x; 8.6150x over previous
import functools

import jax
import jax.numpy as jnp
from jax import lax
from jax.experimental import pallas as pl
from jax.experimental.pallas import tpu as pltpu
from jax.experimental.pallas import tpu_sc as plsc

D_MODEL = 128
N_ROWS = 16384 * 200
NC, NS, L = 2, 16, 16
NW = NC * NS
ROWS_PER_W = N_ROWS // NW
CHUNK = 256
K = CHUNK // 128
N_CHUNKS = ROWS_PER_W // CHUNK


def _build_xidx(xidx, base):
    lane8 = lax.iota(jnp.int32, L) * 8
    for m in range(CHUNK // L):
        xidx[pl.ds(m * L, L)] = lane8 + ((base + m * L) * 8 + 7)


def _start_xgather(x_hbm, xidx, fstage, sem):
    for j in range(K):
        pltpu.async_copy(
            x_hbm.at[xidx.at[pl.ds(j * 128, 128)]],
            fstage.at[pl.ds(j * 128, 128)], sem)


def _wait_xgather(x_hbm, xidx, fstage, sem):
    for j in range(K):
        pltpu.make_async_copy(
            x_hbm.at[xidx.at[pl.ds(j * 128, 128)]],
            fstage.at[pl.ds(j * 128, 128)], sem).wait()


def _start_tgather(table_hbm, idxb, rowbuf, sem):
    for j in range(K):
        pltpu.async_copy(
            table_hbm.at[idxb.at[pl.ds(j * 128, 128)]],
            rowbuf.at[pl.ds(j * 128, 128)], sem)


def _wait_tgather(table_hbm, idxb, rowbuf, sem):
    for j in range(K):
        pltpu.make_async_copy(
            table_hbm.at[idxb.at[pl.ds(j * 128, 128)]],
            rowbuf.at[pl.ds(j * 128, 128)], sem).wait()


def _sc_lookup(x_hbm, table_hbm, out_hbm,
               table_v, xidx0, xidx1, fst0, fst1, idx0, idx1, row0, row1,
               semx0, semx1, semt0, semt1, semw0, semw1):
    wid = lax.axis_index("s") * NC + lax.axis_index("c")
    w_base = wid * ROWS_PER_W
    @pl.when(lax.axis_index("s") == 0)
    def _():
        pltpu.sync_copy(table_hbm, table_v)

    plsc.subcore_barrier()
    table_hbm = table_v
    xidx = [xidx0, xidx1]
    fst = [fst0, fst1]
    idxb = [idx0, idx1]
    row = [row0, row1]
    semx = [semx0, semx1]
    semt = [semt0, semt1]
    semw = [semw0, semw1]

    def halfstep(g, p, first, last):
        base = w_base + g * CHUNK
        q = 1 - p
        _wait_xgather(x_hbm, xidx[p], fst[p], semx[p])
        for i in range(CHUNK // L):
            v = fst[p][pl.ds(i * L, L)]
            idxb[p][pl.ds(i * L, L)] = v.astype(jnp.int32)
        if not first:
            pltpu.make_async_copy(
                row[p], out_hbm.at[pl.ds(base - 2 * CHUNK, CHUNK)],
                semw[p]).wait()
        _start_tgather(table_hbm, idxb[p], row[p], semt[p])
        if not last:
            nbase = w_base + (g + 1) * CHUNK
            _build_xidx(xidx[q], nbase)
            _start_xgather(x_hbm, xidx[q], fst[q], semx[q])
        _wait_tgather(table_hbm, idxb[p], row[p], semt[p])
        pltpu.async_copy(row[p], out_hbm.at[pl.ds(base, CHUNK)], semw[p])

    _build_xidx(xidx[0], w_base)
    _start_xgather(x_hbm, xidx[0], fst[0], semx[0])

    def pair_body(t, carry):
        g0 = t * 2

        @pl.when(t == 0)
        def _():
            halfstep(g0, 0, True, False)
            halfstep(g0 + 1, 1, True, False)

        @pl.when(jnp.logical_and(t > 0, t < N_CHUNKS // 2 - 1))
        def _():
            halfstep(g0, 0, False, False)
            halfstep(g0 + 1, 1, False, False)

        @pl.when(t == N_CHUNKS // 2 - 1)
        def _():
            halfstep(g0, 0, False, False)
            halfstep(g0 + 1, 1, False, True)

        return carry

    lax.fori_loop(0, N_CHUNKS // 2, pair_body, 0)

    endb = w_base + N_CHUNKS * CHUNK
    pltpu.make_async_copy(
        row[0], out_hbm.at[pl.ds(endb - 2 * CHUNK, CHUNK)], semw[0]).wait()
    pltpu.make_async_copy(
        row[1], out_hbm.at[pl.ds(endb - CHUNK, CHUNK)], semw[1]).wait()


def kernel(x, table):
    x_flat = x.reshape(N_ROWS * 8)
    mesh = plsc.VectorSubcoreMesh(core_axis_name="c", subcore_axis_name="s")
    f = functools.partial(
        pl.kernel,
        mesh=mesh,
        out_type=jax.ShapeDtypeStruct((N_ROWS, D_MODEL), jnp.float32),
        scratch_types=[
            pltpu.VMEM_SHARED((10, D_MODEL), jnp.float32),
            pltpu.VMEM((CHUNK,), jnp.int32),
            pltpu.VMEM((CHUNK,), jnp.int32),
            pltpu.VMEM((CHUNK,), jnp.float32),
            pltpu.VMEM((CHUNK,), jnp.float32),
            pltpu.VMEM((CHUNK,), jnp.int32),
            pltpu.VMEM((CHUNK,), jnp.int32),
            pltpu.VMEM((CHUNK, D_MODEL), jnp.float32),
            pltpu.VMEM((CHUNK, D_MODEL), jnp.float32),
            pltpu.SemaphoreType.DMA,
            pltpu.SemaphoreType.DMA,
            pltpu.SemaphoreType.DMA,
            pltpu.SemaphoreType.DMA,
            pltpu.SemaphoreType.DMA,
            pltpu.SemaphoreType.DMA,
        ],
    )(_sc_lookup)
    out = f(x_flat, table)
    return out.reshape(x.shape[0], x.shape[1], D_MODEL)
